# baseline (speedup 1.0000x reference)
import functools

import jax
import jax.numpy as jnp
from jax import lax
from jax.experimental import pallas as pl
from jax.experimental.pallas import tpu as pltpu
from jax.experimental.pallas import tpu_sc as plsc

N = 10000
N2 = 10240
D_IN = 128
H = 256
HH = 128
HEADS = 4
DH = 64
E = 320000
NUM_GRAPHS = 64
EPS = 1e-5
SENT = 1 << 30

TR = 1024
NRT = N2 // TR
TQ = 256
NQ = N2 // TQ
TK = 256

NSUB = 16
C = 80
NCH = 252
EPT = NCH * C
EPAD = NSUB * EPT
RPT = N2 // NSUB
ZR = 16


def _mm(a, b):
    return lax.dot_general(a.astype(jnp.bfloat16), b.astype(jnp.bfloat16),
                           (((1,), (0,)), ((), ())),
                           preferred_element_type=jnp.float32)


def _mm_t(a, b):
    return lax.dot_general(a.astype(jnp.bfloat16), b.astype(jnp.bfloat16),
                           (((1,), (1,)), ((), ())),
                           preferred_element_type=jnp.float32)


def _mm_hi(a, b):
    return lax.dot_general(a, b, (((1,), (0,)), ((), ())),
                           precision=lax.Precision.HIGHEST,
                           preferred_element_type=jnp.float32)


def _in_proj_body(x_ref, w_ref, b_ref, o_ref):
    o_ref[...] = _mm(x_ref[...], w_ref[...]) + b_ref[...]


def _in_proj(x, w, b):
    return pl.pallas_call(
        _in_proj_body,
        grid=(NRT,),
        in_specs=[pl.BlockSpec((TR, D_IN), lambda r: (r, 0)),
                  pl.BlockSpec((D_IN, H), lambda r: (0, 0)),
                  pl.BlockSpec((1, H), lambda r: (0, 0))],
        out_specs=pl.BlockSpec((TR, H), lambda r: (r, 0)),
        out_shape=jax.ShapeDtypeStruct((N2, H), jnp.float32),
    )(x, w, b)


def _ab_body(h_ref, ws_ref, wd_ref, bs_ref, bd_ref, o_ref):
    a = _mm(h_ref[...], ws_ref[...]) + bs_ref[...]
    b = _mm(h_ref[...], wd_ref[...]) + bd_ref[...]
    o_ref[0] = a[:, :HH]
    o_ref[1] = a[:, HH:]
    o_ref[2] = b[:, :HH]
    o_ref[3] = b[:, HH:]


def _ab_proj(h, ws, wd, bs, bd):
    return pl.pallas_call(
        _ab_body,
        grid=(NRT,),
        in_specs=[pl.BlockSpec((TR, H), lambda r: (r, 0)),
                  pl.BlockSpec((H, H), lambda r: (0, 0)),
                  pl.BlockSpec((H, H), lambda r: (0, 0)),
                  pl.BlockSpec((1, H), lambda r: (0, 0)),
                  pl.BlockSpec((1, H), lambda r: (0, 0))],
        out_specs=pl.BlockSpec((4, TR, HH), lambda r: (0, r, 0)),
        out_shape=jax.ShapeDtypeStruct((4, N2, HH), jnp.float32),
    )(h, ws, wd, bs, bd)


def _post_body(agg_ref, ab_ref, h_ref, g_ref, bt_ref, o_ref):
    s0 = jnp.maximum(ab_ref[0] + ab_ref[2], 0.0)
    s1 = jnp.maximum(ab_ref[1] + ab_ref[3], 0.0)
    t0 = agg_ref[0] + s0 + h_ref[:, :HH]
    t1 = agg_ref[1] + s1 + h_ref[:, HH:]
    mean = (jnp.sum(t0, 1, keepdims=True) + jnp.sum(t1, 1, keepdims=True)) / H
    d0 = t0 - mean
    d1 = t1 - mean
    var = (jnp.sum(d0 * d0, 1, keepdims=True)
           + jnp.sum(d1 * d1, 1, keepdims=True)) / H
    ve = var + EPS
    inv = lax.rsqrt(ve)
    inv = inv * (1.5 - 0.5 * ve * inv * inv)
    o_ref[:, :HH] = d0 * inv * g_ref[0, :HH] + bt_ref[0, :HH]
    o_ref[:, HH:] = d1 * inv * g_ref[0, HH:] + bt_ref[0, HH:]


def _post(agg, ab, h, g, bt):
    return pl.pallas_call(
        _post_body,
        grid=(NRT,),
        in_specs=[pl.BlockSpec((2, TR, HH), lambda r: (0, r, 0)),
                  pl.BlockSpec((4, TR, HH), lambda r: (0, r, 0)),
                  pl.BlockSpec((TR, H), lambda r: (r, 0)),
                  pl.BlockSpec((1, H), lambda r: (0, 0)),
                  pl.BlockSpec((1, H), lambda r: (0, 0))],
        out_specs=pl.BlockSpec((TR, H), lambda r: (r, 0)),
        out_shape=jax.ShapeDtypeStruct((N2, H), jnp.float32),
    )(agg, ab, h, g, bt)


def _qkv_body(h_ref, w_ref, b_ref, o_ref):
    qkv = _mm(h_ref[...], w_ref[...]) + b_ref[...]
    for j in range(3 * HEADS):
        o_ref[j] = qkv[:, j * DH:(j + 1) * DH]


def _qkv_proj(h, w, b):
    return pl.pallas_call(
        _qkv_body,
        grid=(NRT,),
        in_specs=[pl.BlockSpec((TR, H), lambda r: (r, 0)),
                  pl.BlockSpec((H, 3 * H), lambda r: (0, 0)),
                  pl.BlockSpec((1, 3 * H), lambda r: (0, 0))],
        out_specs=pl.BlockSpec((3 * HEADS, TR, DH), lambda r: (0, r, 0)),
        out_shape=jax.ShapeDtypeStruct((3 * HEADS, N2, DH), jnp.float32),
    )(h, w, b)


def _attn_body(b_ref, q_ref, k_ref, v_ref, o_ref):
    qt = pl.program_id(1)
    q = q_ref[0] * (1.0 / 8.0)
    bq = b_ref[0, pl.ds(qt * TQ, TQ)]
    ball = b_ref[0, :]
    b0 = bq[0]
    bl = bq[TQ - 1]
    kstart = jnp.sum((ball < b0).astype(jnp.int32))
    kend = jnp.sum((ball <= bl).astype(jnp.int32))
    kb = kstart // TK
    nk = (kend - kb * TK + TK - 1) // TK

    def stile(kj):
        koff = (kb + kj) * TK
        kt = k_ref[0, pl.ds(koff, TK), :]
        bk = b_ref[0, pl.ds(koff, TK)]
        s = _mm_t(q, kt)
        return jnp.where(bq[:, None] == bk[None, :], s, -1e30), koff

    def pass1(kj, carry):
        m, l = carry
        s, _ = stile(kj)
        m2 = jnp.maximum(m, jnp.max(s, axis=1, keepdims=True))
        l2 = l * jnp.exp(m - m2) + jnp.sum(jnp.exp(s - m2), axis=1,
                                           keepdims=True)
        return m2, l2

    m0 = jnp.full((TQ, 1), -1e30, jnp.float32)
    l0 = jnp.zeros((TQ, 1), jnp.float32)
    m, l = lax.fori_loop(0, nk, pass1, (m0, l0))

    def pass2(kj, acc):
        s, koff = stile(kj)
        a = jnp.exp(s - m) / l
        vt = v_ref[0, pl.ds(koff, TK), :]
        return acc + _mm(a, vt)

    acc = lax.fori_loop(0, nk, pass2, jnp.zeros((TQ, DH), jnp.float32))
    o_ref[0] = acc


def _attn(bp, qkv):
    return pl.pallas_call(
        _attn_body,
        grid=(HEADS, NQ),
        in_specs=[pl.BlockSpec((1, N2), lambda h, qt: (0, 0)),
                  pl.BlockSpec((1, TQ, DH), lambda h, qt: (h, qt, 0)),
                  pl.BlockSpec((1, N2, DH), lambda h, qt: (HEADS + h, 0, 0)),
                  pl.BlockSpec((1, N2, DH), lambda h, qt: (2 * HEADS + h, 0, 0))],
        out_specs=pl.BlockSpec((1, TQ, DH), lambda h, qt: (h, qt, 0)),
        out_shape=jax.ShapeDtypeStruct((HEADS, N2, DH), jnp.float32),
    )(bp, qkv, qkv, qkv)


def _oproj_body(o_ref, w_ref, b_ref, out_ref):
    ocat = jnp.concatenate([o_ref[i] for i in range(HEADS)], axis=-1)
    out_ref[...] = _mm(ocat, w_ref[...]) + b_ref[...]


def _o_proj(o, w, b):
    return pl.pallas_call(
        _oproj_body,
        grid=(NRT,),
        in_specs=[pl.BlockSpec((HEADS, TR, DH), lambda r: (0, r, 0)),
                  pl.BlockSpec((H, H), lambda r: (0, 0)),
                  pl.BlockSpec((1, H), lambda r: (0, 0))],
        out_specs=pl.BlockSpec((TR, H), lambda r: (r, 0)),
        out_shape=jax.ShapeDtypeStruct((N2, H), jnp.float32),
    )(o, w, b)


def _pool_body(o_ref, b_ref, w1_ref, b1_ref, w2_ref, b2_ref, out_ref):
    ids = lax.broadcasted_iota(jnp.int32, (NUM_GRAPHS, N2), 0)
    ind = (ids == b_ref[...]).astype(jnp.float32)
    counts = jnp.sum(ind, axis=1, keepdims=True)
    sums = _mm_hi(ind, o_ref[...])
    xg = sums / counts
    hidden = jnp.maximum(_mm(xg, w1_ref[...]) + b1_ref[...], 0.0)
    out_ref[...] = _mm(hidden, w2_ref[...]) + b2_ref[...]


def _pool(o, bp, w1, b1, w2p, b2p):
    return pl.pallas_call(
        _pool_body,
        in_specs=[pl.BlockSpec((N2, H), lambda: (0, 0)),
                  pl.BlockSpec((1, N2), lambda: (0, 0)),
                  pl.BlockSpec((H, HH), lambda: (0, 0)),
                  pl.BlockSpec((1, HH), lambda: (0, 0)),
                  pl.BlockSpec((HH, HH), lambda: (0, 0)),
                  pl.BlockSpec((1, HH), lambda: (0, 0))],
        out_specs=pl.BlockSpec((NUM_GRAPHS, HH), lambda: (0, 0)),
        out_shape=jax.ShapeDtypeStruct((NUM_GRAPHS, HH), jnp.float32),
    )(o, bp, w1, b1, w2p, b2p)


def _edge_body(ab_hbm, gsrc_hbm, gdst_hbm, dst_hbm, out_hbm,
               ia0, ia1, ib0, ib1, dv0, dv1,
               ra0, ra1, rb0, rb1, zbuf, aggsh,
               si0, si1, sg0, sg1):
    c = lax.axis_index("c")
    s = lax.axis_index("s")
    idxa = (ia0, ia1)
    idxb = (ib0, ib1)
    dstv = (dv0, dv1)
    rowsa = (ra0, ra1)
    rowsb = (rb0, rb1)
    si = (si0, si1)
    sg = (sg0, sg1)

    def zrow(i, _):
        for j in range(HH // 16):
            zbuf[i, pl.ds(j * 16, 16)] = jnp.zeros((16,), jnp.float32)
        return 0
    lax.fori_loop(0, ZR, zrow, 0)

    def zcp(i, _):
        pltpu.sync_copy(zbuf, aggsh.at[pl.ds(s * RPT + i * ZR, ZR)])
        return 0
    lax.fori_loop(0, RPT // ZR, zcp, 0)
    plsc.subcore_barrier()

    base = s * EPT

    def idx_copies(i, slot):
        return (
            pltpu.make_async_copy(gsrc_hbm.at[pl.ds(c * EPAD + base + i * C, C)],
                                  idxa[slot], si[slot]),
            pltpu.make_async_copy(gdst_hbm.at[pl.ds(c * EPAD + base + i * C, C)],
                                  idxb[slot], si[slot]),
            pltpu.make_async_copy(dst_hbm.at[pl.ds(base + i * C, C)],
                                  dstv[slot], si[slot]),
        )

    def gath_copies(i, slot):
        return (
            pltpu.make_async_copy(ab_hbm.at[idxa[slot]], rowsa[slot], sg[slot]),
            pltpu.make_async_copy(ab_hbm.at[idxb[slot]], rowsb[slot], sg[slot]),
        )

    def fire_idx(i, slot):
        for cp in idx_copies(i, slot):
            cp.start()

    def wait_idx(i, slot):
        for cp in idx_copies(i, slot):
            cp.wait()

    def fire_gath(i, slot):
        for cp in gath_copies(i, slot):
            cp.start()

    def wait_gath(i, slot):
        for cp in gath_copies(i, slot):
            cp.wait()

    def consume(slot):
        ra = rowsa[slot]
        rb = rowsb[slot]

        def erow(e, _):
            for j in range(HH // 16):
                sl = pl.ds(j * 16, 16)
                rb[e, sl] = jnp.maximum(ra[e, sl] + rb[e, sl], 0.0)
            return 0
        lax.fori_loop(0, C, erow, 0)
        pltpu.sync_copy(rb, aggsh.at[dstv[slot]], add=True)

    fire_idx(0, 0)
    wait_idx(0, 0)
    fire_gath(0, 0)
    fire_idx(1, 1)

    def pair(g, _):
        i0 = 2 * g
        i1 = 2 * g + 1
        i2 = jnp.minimum(i1 + 1, NCH - 1)
        i3 = jnp.minimum(i1 + 2, NCH - 1)
        wait_idx(i1, 1)
        fire_gath(i1, 1)
        wait_gath(i0, 0)
        consume(0)
        fire_idx(i2, 0)
        wait_gath(i1, 1)
        consume(1)
        wait_idx(i2, 0)
        fire_gath(i2, 0)
        fire_idx(i3, 1)
        return 0
    lax.fori_loop(0, NCH // 2, pair, 0)
    wait_gath(NCH - 1, 0)
    wait_idx(NCH - 1, 1)

    plsc.subcore_barrier()
    pltpu.sync_copy(aggsh.at[pl.ds(s * RPT, RPT)],
                    out_hbm.at[pl.ds(c * N2 + s * RPT, RPT)])


def _edge_aggregate(ab_flat, gsrc, gdst, dst):
    mesh = plsc.VectorSubcoreMesh(core_axis_name="c", subcore_axis_name="s")
    k = functools.partial(
        pl.kernel,
        out_type=jax.ShapeDtypeStruct((2 * N2, HH), jnp.float32),
        mesh=mesh,
        scratch_types=[
            pltpu.VMEM((C,), jnp.int32),
            pltpu.VMEM((C,), jnp.int32),
            pltpu.VMEM((C,), jnp.int32),
            pltpu.VMEM((C,), jnp.int32),
            pltpu.VMEM((C,), jnp.int32),
            pltpu.VMEM((C,), jnp.int32),
            pltpu.VMEM((C, HH), jnp.float32),
            pltpu.VMEM((C, HH), jnp.float32),
            pltpu.VMEM((C, HH), jnp.float32),
            pltpu.VMEM((C, HH), jnp.float32),
            pltpu.VMEM((ZR, HH), jnp.float32),
            pltpu.VMEM_SHARED((N2, HH), jnp.float32),
            pltpu.SemaphoreType.DMA,
            pltpu.SemaphoreType.DMA,
            pltpu.SemaphoreType.DMA,
            pltpu.SemaphoreType.DMA,
        ],
    )(_edge_body)
    return k(ab_flat, gsrc, gdst, dst)


def kernel(x, edge_index, batch, params):
    x = x.astype(jnp.float32)
    src = edge_index[0].astype(jnp.int32)
    dst = edge_index[1].astype(jnp.int32)
    batch = batch.astype(jnp.int32)

    xp = jnp.pad(x, ((0, N2 - N), (0, 0)))
    bp = jnp.pad(batch, (0, N2 - N), constant_values=SENT).reshape(1, N2)
    srcp = jnp.pad(src, (0, EPAD - E))
    dstp = jnp.pad(dst, (0, EPAD - E), constant_values=N2 - 1)
    gsrc = jnp.concatenate([srcp, srcp + N2])
    gdst = jnp.concatenate([dstp + 2 * N2, dstp + 3 * N2])

    p = params
    h = _in_proj(xp, p['W_in'], p['b_in'].reshape(1, H))
    for lp in p['layers']:
        ab = _ab_proj(h, lp['W_src'], lp['W_dst'],
                      lp['b_src'].reshape(1, H), lp['b_dst'].reshape(1, H))
        agg = _edge_aggregate(ab.reshape(4 * N2, HH), gsrc, gdst, dstp)
        h = _post(agg.reshape(2, N2, HH), ab, h,
                  lp['gamma'].reshape(1, H), lp['beta'].reshape(1, H))

    qkv = _qkv_proj(h, p['Wqkv'], p['bqkv'].reshape(1, 3 * H))
    o = _attn(bp, qkv)
    hat = _o_proj(o, p['Wo'], p['bo'].reshape(1, H))

    w2p = jnp.pad(p['W_c2'], ((0, 0), (0, HH - 2)))
    b2p = jnp.pad(p['b_c2'], (0, HH - 2)).reshape(1, HH)
    lg = _pool(hat, bp, p['W_c1'], p['b_c1'].reshape(1, HH), w2p, b2p)
    return lg[:, :2]


# speedup vs baseline: 1.1239x; 1.1239x over previous
import functools

import jax
import jax.numpy as jnp
from jax import lax
from jax.experimental import pallas as pl
from jax.experimental.pallas import tpu as pltpu
from jax.experimental.pallas import tpu_sc as plsc

N = 10000
N2 = 10240
D_IN = 128
H = 256
HH = 128
HEADS = 4
DH = 64
E = 320000
NUM_GRAPHS = 64
EPS = 1e-5
SENT = 1 << 30

TR = 1024
NRT = N2 // TR
TQ = 256
NQ = N2 // TQ
TK = 256

NSUB = 16
C = 32
NCH = 626
EPT = NCH * C
EPAD = NSUB * EPT
RPT = N2 // NSUB
ZR = 16


def _mm(a, b):
    return lax.dot_general(a.astype(jnp.bfloat16), b.astype(jnp.bfloat16),
                           (((1,), (0,)), ((), ())),
                           preferred_element_type=jnp.float32)


def _mm_t(a, b):
    return lax.dot_general(a.astype(jnp.bfloat16), b.astype(jnp.bfloat16),
                           (((1,), (1,)), ((), ())),
                           preferred_element_type=jnp.float32)


def _mm_hi(a, b):
    return lax.dot_general(a, b, (((1,), (0,)), ((), ())),
                           precision=lax.Precision.HIGHEST,
                           preferred_element_type=jnp.float32)


def _in_proj_body(x_ref, w_ref, b_ref, o_ref):
    o_ref[...] = _mm(x_ref[...], w_ref[...]) + b_ref[...]


def _in_proj(x, w, b):
    return pl.pallas_call(
        _in_proj_body,
        grid=(NRT,),
        in_specs=[pl.BlockSpec((TR, D_IN), lambda r: (r, 0)),
                  pl.BlockSpec((D_IN, H), lambda r: (0, 0)),
                  pl.BlockSpec((1, H), lambda r: (0, 0))],
        out_specs=pl.BlockSpec((TR, H), lambda r: (r, 0)),
        out_shape=jax.ShapeDtypeStruct((N2, H), jnp.float32),
    )(x, w, b)


def _ab_body(h_ref, ws_ref, wd_ref, bs_ref, bd_ref, o_ref):
    a = _mm(h_ref[...], ws_ref[...]) + bs_ref[...]
    b = _mm(h_ref[...], wd_ref[...]) + bd_ref[...]
    o_ref[0] = a[:, :HH]
    o_ref[1] = a[:, HH:]
    o_ref[2] = b[:, :HH]
    o_ref[3] = b[:, HH:]


def _ab_proj(h, ws, wd, bs, bd):
    return pl.pallas_call(
        _ab_body,
        grid=(NRT,),
        in_specs=[pl.BlockSpec((TR, H), lambda r: (r, 0)),
                  pl.BlockSpec((H, H), lambda r: (0, 0)),
                  pl.BlockSpec((H, H), lambda r: (0, 0)),
                  pl.BlockSpec((1, H), lambda r: (0, 0)),
                  pl.BlockSpec((1, H), lambda r: (0, 0))],
        out_specs=pl.BlockSpec((4, TR, HH), lambda r: (0, r, 0)),
        out_shape=jax.ShapeDtypeStruct((4, N2, HH), jnp.float32),
    )(h, ws, wd, bs, bd)


def _post_body(agg_ref, ab_ref, h_ref, g_ref, bt_ref, o_ref):
    s0 = jnp.maximum(ab_ref[0] + ab_ref[2], 0.0)
    s1 = jnp.maximum(ab_ref[1] + ab_ref[3], 0.0)
    t0 = agg_ref[0] + s0 + h_ref[:, :HH]
    t1 = agg_ref[1] + s1 + h_ref[:, HH:]
    mean = (jnp.sum(t0, 1, keepdims=True) + jnp.sum(t1, 1, keepdims=True)) / H
    d0 = t0 - mean
    d1 = t1 - mean
    var = (jnp.sum(d0 * d0, 1, keepdims=True)
           + jnp.sum(d1 * d1, 1, keepdims=True)) / H
    ve = var + EPS
    inv = lax.rsqrt(ve)
    inv = inv * (1.5 - 0.5 * ve * inv * inv)
    o_ref[:, :HH] = d0 * inv * g_ref[0, :HH] + bt_ref[0, :HH]
    o_ref[:, HH:] = d1 * inv * g_ref[0, HH:] + bt_ref[0, HH:]


def _post(agg, ab, h, g, bt):
    return pl.pallas_call(
        _post_body,
        grid=(NRT,),
        in_specs=[pl.BlockSpec((2, TR, HH), lambda r: (0, r, 0)),
                  pl.BlockSpec((4, TR, HH), lambda r: (0, r, 0)),
                  pl.BlockSpec((TR, H), lambda r: (r, 0)),
                  pl.BlockSpec((1, H), lambda r: (0, 0)),
                  pl.BlockSpec((1, H), lambda r: (0, 0))],
        out_specs=pl.BlockSpec((TR, H), lambda r: (r, 0)),
        out_shape=jax.ShapeDtypeStruct((N2, H), jnp.float32),
    )(agg, ab, h, g, bt)


def _qkv_body(h_ref, w_ref, b_ref, o_ref):
    qkv = _mm(h_ref[...], w_ref[...]) + b_ref[...]
    for j in range(3 * HEADS):
        o_ref[j] = qkv[:, j * DH:(j + 1) * DH]


def _qkv_proj(h, w, b):
    return pl.pallas_call(
        _qkv_body,
        grid=(NRT,),
        in_specs=[pl.BlockSpec((TR, H), lambda r: (r, 0)),
                  pl.BlockSpec((H, 3 * H), lambda r: (0, 0)),
                  pl.BlockSpec((1, 3 * H), lambda r: (0, 0))],
        out_specs=pl.BlockSpec((3 * HEADS, TR, DH), lambda r: (0, r, 0)),
        out_shape=jax.ShapeDtypeStruct((3 * HEADS, N2, DH), jnp.float32),
    )(h, w, b)


def _attn_body(b_ref, q_ref, k_ref, v_ref, o_ref):
    qt = pl.program_id(1)
    q = q_ref[0] * (1.0 / 8.0)
    bq = b_ref[0, pl.ds(qt * TQ, TQ)]
    ball = b_ref[0, :]
    b0 = bq[0]
    bl = bq[TQ - 1]
    kstart = jnp.sum((ball < b0).astype(jnp.int32))
    kend = jnp.sum((ball <= bl).astype(jnp.int32))
    kb = kstart // TK
    nk = (kend - kb * TK + TK - 1) // TK

    def stile(kj):
        koff = (kb + kj) * TK
        kt = k_ref[0, pl.ds(koff, TK), :]
        bk = b_ref[0, pl.ds(koff, TK)]
        s = _mm_t(q, kt)
        return jnp.where(bq[:, None] == bk[None, :], s, -1e30), koff

    def pass1(kj, carry):
        m, l = carry
        s, _ = stile(kj)
        m2 = jnp.maximum(m, jnp.max(s, axis=1, keepdims=True))
        l2 = l * jnp.exp(m - m2) + jnp.sum(jnp.exp(s - m2), axis=1,
                                           keepdims=True)
        return m2, l2

    m0 = jnp.full((TQ, 1), -1e30, jnp.float32)
    l0 = jnp.zeros((TQ, 1), jnp.float32)
    m, l = lax.fori_loop(0, nk, pass1, (m0, l0))

    def pass2(kj, acc):
        s, koff = stile(kj)
        a = jnp.exp(s - m) / l
        vt = v_ref[0, pl.ds(koff, TK), :]
        return acc + _mm(a, vt)

    acc = lax.fori_loop(0, nk, pass2, jnp.zeros((TQ, DH), jnp.float32))
    o_ref[0] = acc


def _attn(bp, qkv):
    return pl.pallas_call(
        _attn_body,
        grid=(HEADS, NQ),
        in_specs=[pl.BlockSpec((1, N2), lambda h, qt: (0, 0)),
                  pl.BlockSpec((1, TQ, DH), lambda h, qt: (h, qt, 0)),
                  pl.BlockSpec((1, N2, DH), lambda h, qt: (HEADS + h, 0, 0)),
                  pl.BlockSpec((1, N2, DH), lambda h, qt: (2 * HEADS + h, 0, 0))],
        out_specs=pl.BlockSpec((1, TQ, DH), lambda h, qt: (h, qt, 0)),
        out_shape=jax.ShapeDtypeStruct((HEADS, N2, DH), jnp.float32),
    )(bp, qkv, qkv, qkv)


def _oproj_body(o_ref, w_ref, b_ref, out_ref):
    ocat = jnp.concatenate([o_ref[i] for i in range(HEADS)], axis=-1)
    out_ref[...] = _mm(ocat, w_ref[...]) + b_ref[...]


def _o_proj(o, w, b):
    return pl.pallas_call(
        _oproj_body,
        grid=(NRT,),
        in_specs=[pl.BlockSpec((HEADS, TR, DH), lambda r: (0, r, 0)),
                  pl.BlockSpec((H, H), lambda r: (0, 0)),
                  pl.BlockSpec((1, H), lambda r: (0, 0))],
        out_specs=pl.BlockSpec((TR, H), lambda r: (r, 0)),
        out_shape=jax.ShapeDtypeStruct((N2, H), jnp.float32),
    )(o, w, b)


def _pool_body(o_ref, b_ref, w1_ref, b1_ref, w2_ref, b2_ref, out_ref):
    ids = lax.broadcasted_iota(jnp.int32, (NUM_GRAPHS, N2), 0)
    ind = (ids == b_ref[...]).astype(jnp.float32)
    counts = jnp.sum(ind, axis=1, keepdims=True)
    sums = _mm_hi(ind, o_ref[...])
    xg = sums / counts
    hidden = jnp.maximum(_mm(xg, w1_ref[...]) + b1_ref[...], 0.0)
    out_ref[...] = _mm(hidden, w2_ref[...]) + b2_ref[...]


def _pool(o, bp, w1, b1, w2p, b2p):
    return pl.pallas_call(
        _pool_body,
        in_specs=[pl.BlockSpec((N2, H), lambda: (0, 0)),
                  pl.BlockSpec((1, N2), lambda: (0, 0)),
                  pl.BlockSpec((H, HH), lambda: (0, 0)),
                  pl.BlockSpec((1, HH), lambda: (0, 0)),
                  pl.BlockSpec((HH, HH), lambda: (0, 0)),
                  pl.BlockSpec((1, HH), lambda: (0, 0))],
        out_specs=pl.BlockSpec((NUM_GRAPHS, HH), lambda: (0, 0)),
        out_shape=jax.ShapeDtypeStruct((NUM_GRAPHS, HH), jnp.float32),
    )(o, bp, w1, b1, w2p, b2p)


def _edge_body(ab_hbm, gsrc_hbm, gdst_hbm, dst_hbm, out_hbm,
               ia0, ia1, ib0, ib1, dv0, dv1,
               ra0, ra1, rb0, rb1, zbuf, aggsh,
               si0, si1, sg0, sg1):
    c = lax.axis_index("c")
    s = lax.axis_index("s")
    idxa = (ia0, ia1)
    idxb = (ib0, ib1)
    dstv = (dv0, dv1)
    rowsa = (ra0, ra1)
    rowsb = (rb0, rb1)
    si = (si0, si1)
    sg = (sg0, sg1)

    def zrow(i, _):
        for j in range(HH // 16):
            zbuf[i, pl.ds(j * 16, 16)] = jnp.zeros((16,), jnp.float32)
        return 0
    lax.fori_loop(0, ZR, zrow, 0)

    def zcp(i, _):
        pltpu.sync_copy(zbuf, aggsh.at[pl.ds(s * RPT + i * ZR, ZR)])
        return 0
    lax.fori_loop(0, RPT // ZR, zcp, 0)
    plsc.subcore_barrier()

    base = s * EPT

    def idx_copies(i, slot):
        return (
            pltpu.make_async_copy(gsrc_hbm.at[pl.ds(c * EPAD + base + i * C, C)],
                                  idxa[slot], si[slot]),
            pltpu.make_async_copy(gdst_hbm.at[pl.ds(c * EPAD + base + i * C, C)],
                                  idxb[slot], si[slot]),
            pltpu.make_async_copy(dst_hbm.at[pl.ds(base + i * C, C)],
                                  dstv[slot], si[slot]),
        )

    def gath_copies(i, slot):
        return (
            pltpu.make_async_copy(ab_hbm.at[idxa[slot]], rowsa[slot], sg[slot]),
            pltpu.make_async_copy(ab_hbm.at[idxb[slot]], rowsb[slot], sg[slot]),
        )

    def fire_idx(i, slot):
        for cp in idx_copies(i, slot):
            cp.start()

    def wait_idx(i, slot):
        for cp in idx_copies(i, slot):
            cp.wait()

    def fire_gath(i, slot):
        for cp in gath_copies(i, slot):
            cp.start()

    def wait_gath(i, slot):
        for cp in gath_copies(i, slot):
            cp.wait()

    def consume(slot):
        ra = rowsa[slot]
        rb = rowsb[slot]

        def erow(e, _):
            for j in range(HH // 16):
                sl = pl.ds(j * 16, 16)
                rb[e, sl] = jnp.maximum(ra[e, sl] + rb[e, sl], 0.0)
            return 0
        lax.fori_loop(0, C, erow, 0)
        pltpu.sync_copy(rb, aggsh.at[dstv[slot]], add=True)

    fire_idx(0, 0)
    wait_idx(0, 0)
    fire_gath(0, 0)
    fire_idx(1, 1)

    def pair(g, _):
        i0 = 2 * g
        i1 = 2 * g + 1
        i2 = jnp.minimum(i1 + 1, NCH - 1)
        i3 = jnp.minimum(i1 + 2, NCH - 1)
        wait_idx(i1, 1)
        fire_gath(i1, 1)
        wait_gath(i0, 0)
        consume(0)
        fire_idx(i2, 0)
        wait_gath(i1, 1)
        consume(1)
        wait_idx(i2, 0)
        fire_gath(i2, 0)
        fire_idx(i3, 1)
        return 0
    lax.fori_loop(0, NCH // 2, pair, 0)
    wait_gath(NCH - 1, 0)
    wait_idx(NCH - 1, 1)

    plsc.subcore_barrier()
    pltpu.sync_copy(aggsh.at[pl.ds(s * RPT, RPT)],
                    out_hbm.at[pl.ds(c * N2 + s * RPT, RPT)])


def _edge_aggregate(ab_flat, gsrc, gdst, dst):
    mesh = plsc.VectorSubcoreMesh(core_axis_name="c", subcore_axis_name="s")
    k = functools.partial(
        pl.kernel,
        out_type=jax.ShapeDtypeStruct((2 * N2, HH), jnp.float32),
        mesh=mesh,
        scratch_types=[
            pltpu.VMEM((C,), jnp.int32),
            pltpu.VMEM((C,), jnp.int32),
            pltpu.VMEM((C,), jnp.int32),
            pltpu.VMEM((C,), jnp.int32),
            pltpu.VMEM((C,), jnp.int32),
            pltpu.VMEM((C,), jnp.int32),
            pltpu.VMEM((C, HH), jnp.float32),
            pltpu.VMEM((C, HH), jnp.float32),
            pltpu.VMEM((C, HH), jnp.float32),
            pltpu.VMEM((C, HH), jnp.float32),
            pltpu.VMEM((ZR, HH), jnp.float32),
            pltpu.VMEM_SHARED((N2, HH), jnp.float32),
            pltpu.SemaphoreType.DMA,
            pltpu.SemaphoreType.DMA,
            pltpu.SemaphoreType.DMA,
            pltpu.SemaphoreType.DMA,
        ],
    )(_edge_body)
    return k(ab_flat, gsrc, gdst, dst)


def kernel(x, edge_index, batch, params):
    x = x.astype(jnp.float32)
    src = edge_index[0].astype(jnp.int32)
    dst = edge_index[1].astype(jnp.int32)
    batch = batch.astype(jnp.int32)

    xp = jnp.pad(x, ((0, N2 - N), (0, 0)))
    bp = jnp.pad(batch, (0, N2 - N), constant_values=SENT).reshape(1, N2)
    srcp = jnp.pad(src, (0, EPAD - E))
    dstp = jnp.pad(dst, (0, EPAD - E), constant_values=N2 - 1)
    gsrc = jnp.concatenate([srcp, srcp + N2])
    gdst = jnp.concatenate([dstp + 2 * N2, dstp + 3 * N2])

    p = params
    h = _in_proj(xp, p['W_in'], p['b_in'].reshape(1, H))
    for lp in p['layers']:
        ab = _ab_proj(h, lp['W_src'], lp['W_dst'],
                      lp['b_src'].reshape(1, H), lp['b_dst'].reshape(1, H))
        agg = _edge_aggregate(ab.reshape(4 * N2, HH), gsrc, gdst, dstp)
        h = _post(agg.reshape(2, N2, HH), ab, h,
                  lp['gamma'].reshape(1, H), lp['beta'].reshape(1, H))

    qkv = _qkv_proj(h, p['Wqkv'], p['bqkv'].reshape(1, 3 * H))
    o = _attn(bp, qkv)
    hat = _o_proj(o, p['Wo'], p['bo'].reshape(1, H))

    w2p = jnp.pad(p['W_c2'], ((0, 0), (0, HH - 2)))
    b2p = jnp.pad(p['b_c2'], (0, HH - 2)).reshape(1, HH)
    lg = _pool(hat, bp, p['W_c1'], p['b_c1'].reshape(1, HH), w2p, b2p)
    return lg[:, :2]


# speedup vs baseline: 1.2094x; 1.0761x over previous
import functools

import jax
import jax.numpy as jnp
from jax import lax
from jax.experimental import pallas as pl
from jax.experimental.pallas import tpu as pltpu
from jax.experimental.pallas import tpu_sc as plsc

N = 10000
N2 = 10240
D_IN = 128
H = 256
HH = 128
HEADS = 4
DH = 64
E = 320000
NUM_GRAPHS = 64
EPS = 1e-5
SENT = 1 << 30

TR = 1024
NRT = N2 // TR
TQ = 256
NQ = N2 // TQ
TK = 256

NSUB = 16
C = 56
NCH = 360
EPT = NCH * C
EPAD = NSUB * EPT
RPT = N2 // NSUB
ZR = 16


def _mm(a, b):
    return lax.dot_general(a.astype(jnp.bfloat16), b.astype(jnp.bfloat16),
                           (((1,), (0,)), ((), ())),
                           preferred_element_type=jnp.float32)


def _mm_t(a, b):
    return lax.dot_general(a.astype(jnp.bfloat16), b.astype(jnp.bfloat16),
                           (((1,), (1,)), ((), ())),
                           preferred_element_type=jnp.float32)


def _mm_hi(a, b):
    return lax.dot_general(a, b, (((1,), (0,)), ((), ())),
                           precision=lax.Precision.HIGHEST,
                           preferred_element_type=jnp.float32)


def _in_proj_body(x_ref, w_ref, b_ref, o_ref):
    o_ref[...] = _mm(x_ref[...], w_ref[...]) + b_ref[...]


def _in_proj(x, w, b):
    return pl.pallas_call(
        _in_proj_body,
        grid=(NRT,),
        in_specs=[pl.BlockSpec((TR, D_IN), lambda r: (r, 0)),
                  pl.BlockSpec((D_IN, H), lambda r: (0, 0)),
                  pl.BlockSpec((1, H), lambda r: (0, 0))],
        out_specs=pl.BlockSpec((TR, H), lambda r: (r, 0)),
        out_shape=jax.ShapeDtypeStruct((N2, H), jnp.float32),
    )(x, w, b)


def _ab_body(h_ref, ws_ref, wd_ref, bs_ref, bd_ref, o_ref):
    a = _mm(h_ref[...], ws_ref[...]) + bs_ref[...]
    b = _mm(h_ref[...], wd_ref[...]) + bd_ref[...]
    o_ref[0] = a[:, :HH]
    o_ref[1] = a[:, HH:]
    o_ref[2] = b[:, :HH]
    o_ref[3] = b[:, HH:]


def _ab_proj(h, ws, wd, bs, bd):
    return pl.pallas_call(
        _ab_body,
        grid=(NRT,),
        in_specs=[pl.BlockSpec((TR, H), lambda r: (r, 0)),
                  pl.BlockSpec((H, H), lambda r: (0, 0)),
                  pl.BlockSpec((H, H), lambda r: (0, 0)),
                  pl.BlockSpec((1, H), lambda r: (0, 0)),
                  pl.BlockSpec((1, H), lambda r: (0, 0))],
        out_specs=pl.BlockSpec((4, TR, HH), lambda r: (0, r, 0)),
        out_shape=jax.ShapeDtypeStruct((4, N2, HH), jnp.float32),
    )(h, ws, wd, bs, bd)


def _post_body(agg_ref, ab_ref, h_ref, g_ref, bt_ref, o_ref):
    s0 = jnp.maximum(ab_ref[0] + ab_ref[2], 0.0)
    s1 = jnp.maximum(ab_ref[1] + ab_ref[3], 0.0)
    t0 = agg_ref[0] + s0 + h_ref[:, :HH]
    t1 = agg_ref[1] + s1 + h_ref[:, HH:]
    mean = (jnp.sum(t0, 1, keepdims=True) + jnp.sum(t1, 1, keepdims=True)) / H
    d0 = t0 - mean
    d1 = t1 - mean
    var = (jnp.sum(d0 * d0, 1, keepdims=True)
           + jnp.sum(d1 * d1, 1, keepdims=True)) / H
    ve = var + EPS
    inv = lax.rsqrt(ve)
    inv = inv * (1.5 - 0.5 * ve * inv * inv)
    o_ref[:, :HH] = d0 * inv * g_ref[0, :HH] + bt_ref[0, :HH]
    o_ref[:, HH:] = d1 * inv * g_ref[0, HH:] + bt_ref[0, HH:]


def _post(agg, ab, h, g, bt):
    return pl.pallas_call(
        _post_body,
        grid=(NRT,),
        in_specs=[pl.BlockSpec((2, TR, HH), lambda r: (0, r, 0)),
                  pl.BlockSpec((4, TR, HH), lambda r: (0, r, 0)),
                  pl.BlockSpec((TR, H), lambda r: (r, 0)),
                  pl.BlockSpec((1, H), lambda r: (0, 0)),
                  pl.BlockSpec((1, H), lambda r: (0, 0))],
        out_specs=pl.BlockSpec((TR, H), lambda r: (r, 0)),
        out_shape=jax.ShapeDtypeStruct((N2, H), jnp.float32),
    )(agg, ab, h, g, bt)


def _qkv_body(h_ref, w_ref, b_ref, o_ref):
    qkv = _mm(h_ref[...], w_ref[...]) + b_ref[...]
    for j in range(3 * HEADS):
        o_ref[j] = qkv[:, j * DH:(j + 1) * DH]


def _qkv_proj(h, w, b):
    return pl.pallas_call(
        _qkv_body,
        grid=(NRT,),
        in_specs=[pl.BlockSpec((TR, H), lambda r: (r, 0)),
                  pl.BlockSpec((H, 3 * H), lambda r: (0, 0)),
                  pl.BlockSpec((1, 3 * H), lambda r: (0, 0))],
        out_specs=pl.BlockSpec((3 * HEADS, TR, DH), lambda r: (0, r, 0)),
        out_shape=jax.ShapeDtypeStruct((3 * HEADS, N2, DH), jnp.float32),
    )(h, w, b)


def _attn_body(b_ref, q_ref, k_ref, v_ref, o_ref):
    qt = pl.program_id(1)
    q = q_ref[0] * (1.0 / 8.0)
    bq = b_ref[0, pl.ds(qt * TQ, TQ)]
    ball = b_ref[0, :]
    b0 = bq[0]
    bl = bq[TQ - 1]
    kstart = jnp.sum((ball < b0).astype(jnp.int32))
    kend = jnp.sum((ball <= bl).astype(jnp.int32))
    kb = kstart // TK
    nk = (kend - kb * TK + TK - 1) // TK

    def stile(kj):
        koff = (kb + kj) * TK
        kt = k_ref[0, pl.ds(koff, TK), :]
        bk = b_ref[0, pl.ds(koff, TK)]
        s = _mm_t(q, kt)
        return jnp.where(bq[:, None] == bk[None, :], s, -1e30), koff

    def pass1(kj, carry):
        m, l = carry
        s, _ = stile(kj)
        m2 = jnp.maximum(m, jnp.max(s, axis=1, keepdims=True))
        l2 = l * jnp.exp(m - m2) + jnp.sum(jnp.exp(s - m2), axis=1,
                                           keepdims=True)
        return m2, l2

    m0 = jnp.full((TQ, 1), -1e30, jnp.float32)
    l0 = jnp.zeros((TQ, 1), jnp.float32)
    m, l = lax.fori_loop(0, nk, pass1, (m0, l0))

    def pass2(kj, acc):
        s, koff = stile(kj)
        a = jnp.exp(s - m) / l
        vt = v_ref[0, pl.ds(koff, TK), :]
        return acc + _mm(a, vt)

    acc = lax.fori_loop(0, nk, pass2, jnp.zeros((TQ, DH), jnp.float32))
    o_ref[0] = acc


def _attn(bp, qkv):
    return pl.pallas_call(
        _attn_body,
        grid=(HEADS, NQ),
        in_specs=[pl.BlockSpec((1, N2), lambda h, qt: (0, 0)),
                  pl.BlockSpec((1, TQ, DH), lambda h, qt: (h, qt, 0)),
                  pl.BlockSpec((1, N2, DH), lambda h, qt: (HEADS + h, 0, 0)),
                  pl.BlockSpec((1, N2, DH), lambda h, qt: (2 * HEADS + h, 0, 0))],
        out_specs=pl.BlockSpec((1, TQ, DH), lambda h, qt: (h, qt, 0)),
        out_shape=jax.ShapeDtypeStruct((HEADS, N2, DH), jnp.float32),
    )(bp, qkv, qkv, qkv)


def _oproj_body(o_ref, w_ref, b_ref, out_ref):
    ocat = jnp.concatenate([o_ref[i] for i in range(HEADS)], axis=-1)
    out_ref[...] = _mm(ocat, w_ref[...]) + b_ref[...]


def _o_proj(o, w, b):
    return pl.pallas_call(
        _oproj_body,
        grid=(NRT,),
        in_specs=[pl.BlockSpec((HEADS, TR, DH), lambda r: (0, r, 0)),
                  pl.BlockSpec((H, H), lambda r: (0, 0)),
                  pl.BlockSpec((1, H), lambda r: (0, 0))],
        out_specs=pl.BlockSpec((TR, H), lambda r: (r, 0)),
        out_shape=jax.ShapeDtypeStruct((N2, H), jnp.float32),
    )(o, w, b)


def _pool_body(o_ref, b_ref, w1_ref, b1_ref, w2_ref, b2_ref, out_ref):
    ids = lax.broadcasted_iota(jnp.int32, (NUM_GRAPHS, N2), 0)
    ind = (ids == b_ref[...]).astype(jnp.float32)
    counts = jnp.sum(ind, axis=1, keepdims=True)
    sums = _mm_hi(ind, o_ref[...])
    xg = sums / counts
    hidden = jnp.maximum(_mm(xg, w1_ref[...]) + b1_ref[...], 0.0)
    out_ref[...] = _mm(hidden, w2_ref[...]) + b2_ref[...]


def _pool(o, bp, w1, b1, w2p, b2p):
    return pl.pallas_call(
        _pool_body,
        in_specs=[pl.BlockSpec((N2, H), lambda: (0, 0)),
                  pl.BlockSpec((1, N2), lambda: (0, 0)),
                  pl.BlockSpec((H, HH), lambda: (0, 0)),
                  pl.BlockSpec((1, HH), lambda: (0, 0)),
                  pl.BlockSpec((HH, HH), lambda: (0, 0)),
                  pl.BlockSpec((1, HH), lambda: (0, 0))],
        out_specs=pl.BlockSpec((NUM_GRAPHS, HH), lambda: (0, 0)),
        out_shape=jax.ShapeDtypeStruct((NUM_GRAPHS, HH), jnp.float32),
    )(o, bp, w1, b1, w2p, b2p)


def _edge_body(ab_hbm, gsrc_hbm, gdst_hbm, dst_hbm, out_hbm,
               ia0, ia1, ia2, ib0, ib1, ib2, dv0, dv1, dv2,
               ra0, ra1, ra2, rb0, rb1, rb2, zbuf, aggsh,
               si0, si1, si2, sg0, sg1, sg2):
    c = lax.axis_index("c")
    s = lax.axis_index("s")
    idxa = (ia0, ia1, ia2)
    idxb = (ib0, ib1, ib2)
    dstv = (dv0, dv1, dv2)
    rowsa = (ra0, ra1, ra2)
    rowsb = (rb0, rb1, rb2)
    si = (si0, si1, si2)
    sg = (sg0, sg1, sg2)

    def zrow(i, _):
        for j in range(HH // 16):
            zbuf[i, pl.ds(j * 16, 16)] = jnp.zeros((16,), jnp.float32)
        return 0
    lax.fori_loop(0, ZR, zrow, 0)

    def zcp(i, _):
        pltpu.sync_copy(zbuf, aggsh.at[pl.ds(s * RPT + i * ZR, ZR)])
        return 0
    lax.fori_loop(0, RPT // ZR, zcp, 0)
    plsc.subcore_barrier()

    base = s * EPT

    def idx_copies(i, slot):
        return (
            pltpu.make_async_copy(gsrc_hbm.at[pl.ds(c * EPAD + base + i * C, C)],
                                  idxa[slot], si[slot]),
            pltpu.make_async_copy(gdst_hbm.at[pl.ds(c * EPAD + base + i * C, C)],
                                  idxb[slot], si[slot]),
            pltpu.make_async_copy(dst_hbm.at[pl.ds(base + i * C, C)],
                                  dstv[slot], si[slot]),
        )

    def gath_copies(i, slot):
        return (
            pltpu.make_async_copy(ab_hbm.at[idxa[slot]], rowsa[slot], sg[slot]),
            pltpu.make_async_copy(ab_hbm.at[idxb[slot]], rowsb[slot], sg[slot]),
        )

    def fire_idx(i, slot):
        for cp in idx_copies(i, slot):
            cp.start()

    def wait_idx(i, slot):
        for cp in idx_copies(i, slot):
            cp.wait()

    def fire_gath(i, slot):
        for cp in gath_copies(i, slot):
            cp.start()

    def wait_gath(i, slot):
        for cp in gath_copies(i, slot):
            cp.wait()

    def consume(slot):
        ra = rowsa[slot]
        rb = rowsb[slot]

        def erow(e, _):
            for j in range(HH // 16):
                sl = pl.ds(j * 16, 16)
                rb[e, sl] = jnp.maximum(ra[e, sl] + rb[e, sl], 0.0)
            return 0
        lax.fori_loop(0, C, erow, 0)
        pltpu.sync_copy(rb, aggsh.at[dstv[slot]], add=True)

    fire_idx(0, 0)
    wait_idx(0, 0)
    fire_gath(0, 0)
    fire_idx(1, 1)
    wait_idx(1, 1)
    fire_gath(1, 1)
    fire_idx(2, 2)

    def trip(g, _):
        i0 = 3 * g
        i1 = 3 * g + 1
        i2 = 3 * g + 2
        i3 = jnp.minimum(i0 + 3, NCH - 1)
        i4 = jnp.minimum(i0 + 4, NCH - 1)
        i5 = jnp.minimum(i0 + 5, NCH - 1)
        wait_gath(i0, 0)
        consume(0)
        fire_idx(i3, 0)
        wait_idx(i2, 2)
        fire_gath(i2, 2)
        wait_gath(i1, 1)
        consume(1)
        fire_idx(i4, 1)
        wait_idx(i3, 0)
        fire_gath(i3, 0)
        wait_gath(i2, 2)
        consume(2)
        fire_idx(i5, 2)
        wait_idx(i4, 1)
        fire_gath(i4, 1)
        return 0
    lax.fori_loop(0, NCH // 3, trip, 0)
    wait_gath(NCH - 1, 0)
    wait_gath(NCH - 1, 1)
    wait_idx(NCH - 1, 2)

    plsc.subcore_barrier()
    pltpu.sync_copy(aggsh.at[pl.ds(s * RPT, RPT)],
                    out_hbm.at[pl.ds(c * N2 + s * RPT, RPT)])


def _edge_aggregate(ab_flat, gsrc, gdst, dst):
    mesh = plsc.VectorSubcoreMesh(core_axis_name="c", subcore_axis_name="s")
    k = functools.partial(
        pl.kernel,
        out_type=jax.ShapeDtypeStruct((2 * N2, HH), jnp.float32),
        mesh=mesh,
        scratch_types=(
            [pltpu.VMEM((C,), jnp.int32)] * 9
            + [pltpu.VMEM((C, HH), jnp.float32)] * 6
            + [pltpu.VMEM((ZR, HH), jnp.float32),
               pltpu.VMEM_SHARED((N2, HH), jnp.float32)]
            + [pltpu.SemaphoreType.DMA] * 6
        ),
    )(_edge_body)
    return k(ab_flat, gsrc, gdst, dst)


def kernel(x, edge_index, batch, params):
    x = x.astype(jnp.float32)
    src = edge_index[0].astype(jnp.int32)
    dst = edge_index[1].astype(jnp.int32)
    batch = batch.astype(jnp.int32)

    xp = jnp.pad(x, ((0, N2 - N), (0, 0)))
    bp = jnp.pad(batch, (0, N2 - N), constant_values=SENT).reshape(1, N2)
    srcp = jnp.pad(src, (0, EPAD - E))
    dstp = jnp.pad(dst, (0, EPAD - E), constant_values=N2 - 1)
    gsrc = jnp.concatenate([srcp, srcp + N2])
    gdst = jnp.concatenate([dstp + 2 * N2, dstp + 3 * N2])

    p = params
    h = _in_proj(xp, p['W_in'], p['b_in'].reshape(1, H))
    for lp in p['layers']:
        ab = _ab_proj(h, lp['W_src'], lp['W_dst'],
                      lp['b_src'].reshape(1, H), lp['b_dst'].reshape(1, H))
        agg = _edge_aggregate(ab.reshape(4 * N2, HH), gsrc, gdst, dstp)
        h = _post(agg.reshape(2, N2, HH), ab, h,
                  lp['gamma'].reshape(1, H), lp['beta'].reshape(1, H))

    qkv = _qkv_proj(h, p['Wqkv'], p['bqkv'].reshape(1, 3 * H))
    o = _attn(bp, qkv)
    hat = _o_proj(o, p['Wo'], p['bo'].reshape(1, H))

    w2p = jnp.pad(p['W_c2'], ((0, 0), (0, HH - 2)))
    b2p = jnp.pad(p['b_c2'], (0, HH - 2)).reshape(1, HH)
    lg = _pool(hat, bp, p['W_c1'], p['b_c1'].reshape(1, HH), w2p, b2p)
    return lg[:, :2]
